# Initial kernel scaffold; baseline (speedup 1.0000x reference)
import jax
import jax.numpy as jnp
from jax.experimental import pallas as pl


def kernel(x, edge_index, batch, W1, b1, W2, b2, W3, b3, Wl1, bl1, Wl2, bl2):
    raise NotImplementedError("write your pallas kernel here")


# speedup vs baseline: 25.0435x; 25.0435x over previous
import functools

import jax
import jax.numpy as jnp
from jax import lax
from jax.experimental import pallas as pl
from jax.experimental.pallas import tpu as pltpu
from jax.experimental.pallas import tpu_sc as plsc

N = 10000
E = 320000
D = 128
H = 64
G = 64

NC = 2
NS = 16
HW = H // NC
CHUNK = 125
NCHUNK = E // (NS * CHUNK)
DCHUNK = E // (NC * NS * CHUNK)
NBUF = 2
ROWS_PT = N // NS

_mesh = plsc.VectorSubcoreMesh(
    core_axis_name="c", subcore_axis_name="s", num_cores=NC, num_subcores=NS)


def _zero_fill(buf, nrow, ncol):
    z16 = jnp.zeros((16,), jnp.float32)

    @pl.loop(0, nrow)
    def _(i):
        for k in range(ncol // 16):
            buf[i, pl.ds(k * 16, 16)] = z16


@functools.partial(
    pl.kernel,
    out_type=jax.ShapeDtypeStruct((NC, NS, ROWS_PT // CHUNK, CHUNK, 16), jnp.float32),
    mesh=_mesh,
    scratch_types=[
        pltpu.VMEM((DCHUNK, CHUNK), jnp.int32),
        pltpu.VMEM((CHUNK, 16), jnp.float32),
        pltpu.VMEM_SHARED((N, 16), jnp.float32),
        pltpu.SemaphoreType.DMA,
    ],
)
def _sc_degree(dst_hbm, out_hbm, didx, buf, acc, sem):
    c = lax.axis_index("c")
    s = lax.axis_index("s")
    pltpu.sync_copy(dst_hbm.at[c, s], didx)

    _zero_fill(buf, CHUNK, 16)
    for r in range(ROWS_PT // CHUNK):
        pltpu.sync_copy(buf, acc.at[pl.ds((s * (ROWS_PT // CHUNK) + r) * CHUNK, CHUNK)])

    one16 = jnp.ones((16,), jnp.float32)

    @pl.loop(0, CHUNK)
    def _(i):
        buf[i, pl.ds(0, 16)] = one16

    plsc.subcore_barrier()

    @pl.loop(0, DCHUNK, step=8)
    def _(j0):
        descs = [pltpu.async_copy(buf, acc.at[didx.at[j0 + b]], sem, add=True)
                 for b in range(8)]
        for d in descs:
            d.wait()

    plsc.subcore_barrier()
    for r in range(ROWS_PT // CHUNK):
        pltpu.sync_copy(acc.at[pl.ds(s * ROWS_PT + r * CHUNK, CHUNK)], buf)
        pltpu.sync_copy(buf, out_hbm.at[c, s, r])


@functools.partial(
    pl.kernel,
    out_type=jax.ShapeDtypeStruct((NC, NS, ROWS_PT // CHUNK, CHUNK, HW), jnp.float32),
    mesh=_mesh,
    scratch_types=[
        pltpu.VMEM((NCHUNK // 4, CHUNK), jnp.int32),
        pltpu.VMEM((NCHUNK // 4, CHUNK), jnp.int32),
        pltpu.VMEM((NBUF, CHUNK, HW), jnp.float32),
        pltpu.VMEM((CHUNK, HW), jnp.float32),
        pltpu.VMEM_SHARED((N, HW), jnp.float32),
        pltpu.SemaphoreType.DMA,
        pltpu.SemaphoreType.DMA,
    ],
    compiler_params=pltpu.CompilerParams(use_tc_tiling_on_sc=False),
)
def _sc_scatter(y_hbm, src_hbm, dst_hbm, out_hbm,
                sidx, didx, rows, zbuf, acc, gsem, ssem):
    QCH = NCHUNK // 4
    c = lax.axis_index("c")
    s = lax.axis_index("s")

    _zero_fill(zbuf, CHUNK, HW)
    for r in range(ROWS_PT // CHUNK):
        pltpu.sync_copy(zbuf, acc.at[pl.ds((s * (ROWS_PT // CHUNK) + r) * CHUNK, CHUNK)])
    plsc.subcore_barrier()

    for q in range(4):
        pltpu.sync_copy(src_hbm.at[s, pl.ds(q * QCH, QCH)], sidx)
        pltpu.sync_copy(dst_hbm.at[s, pl.ds(q * QCH, QCH)], didx)

        @pl.loop(0, QCH, step=NBUF)
        def _(j0):
            gds = [pltpu.async_copy(y_hbm.at[c].at[sidx.at[j0 + b]], rows.at[b], gsem)
                   for b in range(NBUF)]
            sds = []
            for b in range(NBUF):
                gds[b].wait()
                sds.append(pltpu.async_copy(rows.at[b], acc.at[didx.at[j0 + b]],
                                            ssem, add=True))
            for d in sds:
                d.wait()

    plsc.subcore_barrier()
    for r in range(ROWS_PT // CHUNK):
        pltpu.sync_copy(acc.at[pl.ds(s * ROWS_PT + r * CHUNK, CHUNK)], zbuf)
        pltpu.sync_copy(zbuf, out_hbm.at[c, s, r])


def _split(h):
    return jnp.stack([h[:, :HW], h[:, HW:]], axis=0)


def _tc1_body(cnt2, x, w1, h1t, dinv):
    cnt = cnt2[0, :, 0:1] + cnt2[1, :, 0:1] + 1.0
    di = lax.rsqrt(cnt)
    dinv[...] = di
    h = jnp.dot(x[...], w1[...], preferred_element_type=jnp.float32) * di
    h1t[...] = _split(h)


def _tc2_body(acc1, h1t, dinv, b1, z1t):
    di = dinv[...][None]
    agg = di * (acc1[...] + h1t[...]) + b1[...][:, None, :]
    z1t[...] = jnp.maximum(agg, 0.0) * di


def _tc3_body(acc2, z1t, dinv, w2, b2, w3, h3t):
    di = dinv[...]
    agg2h = di[None] * (acc2[...] + z1t[...])
    agg2 = jnp.concatenate([agg2h[0], agg2h[1]], axis=1)
    z2 = jnp.maximum(jnp.dot(agg2, w2[...], preferred_element_type=jnp.float32)
                     + b2[...][None, :], 0.0)
    h3 = jnp.dot(z2, w3[...], preferred_element_type=jnp.float32) * di
    h3t[...] = _split(h3)


def _tc4_body(acc3, h3t, dinv, b3, batch, wl1, bl1, wl2, bl2, out):
    di = dinv[...][None]
    z3h = jnp.maximum(di * (acc3[...] + h3t[...]) + b3[...][:, None, :], 0.0)
    z3 = jnp.concatenate([z3h[0], z3h[1]], axis=1)
    gids = lax.broadcasted_iota(jnp.int32, (G, N), 0)
    onehot = (batch[...][None, :] == gids).astype(jnp.float32)
    sums = jnp.dot(onehot, z3, preferred_element_type=jnp.float32)
    cnts = jnp.sum(onehot, axis=1, keepdims=True)
    pooled = sums / jnp.maximum(cnts, 1.0)
    z = jnp.maximum(jnp.dot(pooled, wl1[...], preferred_element_type=jnp.float32)
                    + bl1[...][None, :], 0.0)
    out[...] = jnp.dot(z, wl2[...], preferred_element_type=jnp.float32) + bl2[...][None, :]


def _tc_call(body, out_shapes, *args):
    return pl.pallas_call(body, out_shape=out_shapes)(*args)


def kernel(x, edge_index, batch, W1, b1, W2, b2, W3, b3, Wl1, bl1, Wl2, bl2):
    src = edge_index[0].reshape(NS, NCHUNK, CHUNK)
    dst = edge_index[1].reshape(NS, NCHUNK, CHUNK)
    dst_deg = edge_index[1].reshape(NC, NS, DCHUNK, CHUNK)
    b1s = b1.reshape(NC, HW)
    b3s = b3.reshape(NC, HW)

    cnt2 = _sc_degree(dst_deg).reshape(NC, N, 16)
    h1t, dinv = _tc_call(
        _tc1_body,
        (jax.ShapeDtypeStruct((NC, N, HW), jnp.float32),
         jax.ShapeDtypeStruct((N, 1), jnp.float32)),
        cnt2, x, W1)
    acc1 = _sc_scatter(h1t, src, dst).reshape(NC, N, HW)
    z1t = _tc_call(_tc2_body, jax.ShapeDtypeStruct((NC, N, HW), jnp.float32),
                   acc1, h1t, dinv, b1s)
    acc2 = _sc_scatter(z1t, src, dst).reshape(NC, N, HW)
    h3t = _tc_call(_tc3_body, jax.ShapeDtypeStruct((NC, N, HW), jnp.float32),
                   acc2, z1t, dinv, W2, b2, W3)
    acc3 = _sc_scatter(h3t, src, dst).reshape(NC, N, HW)
    out = _tc_call(_tc4_body, jax.ShapeDtypeStruct((G, 1), jnp.float32),
                   acc3, h3t, dinv, b3s, batch, Wl1, bl1, Wl2, bl2)
    return jnp.squeeze(out, -1)


# speedup vs baseline: 30.8647x; 1.2324x over previous
import functools

import jax
import jax.numpy as jnp
from jax import lax
from jax.experimental import pallas as pl
from jax.experimental.pallas import tpu as pltpu
from jax.experimental.pallas import tpu_sc as plsc

N = 10000
E = 320000
D = 128
H = 64
G = 64

NC = 2
NS = 16
HW = H // NC
CHUNK = 125
NCHUNK = E // (NS * CHUNK)
DCHUNK = E // (NC * NS * CHUNK)
NBUF = 4
ROWS_PT = N // NS

_mesh = plsc.VectorSubcoreMesh(
    core_axis_name="c", subcore_axis_name="s", num_cores=NC, num_subcores=NS)


def _zero_fill(buf, nrow, ncol):
    z16 = jnp.zeros((16,), jnp.float32)

    @pl.loop(0, nrow)
    def _(i):
        for k in range(ncol // 16):
            buf[i, pl.ds(k * 16, 16)] = z16


@functools.partial(
    pl.kernel,
    out_type=jax.ShapeDtypeStruct((NC, NS, ROWS_PT // CHUNK, CHUNK, 16), jnp.float32),
    mesh=_mesh,
    scratch_types=[
        pltpu.VMEM((DCHUNK, CHUNK), jnp.int32),
        pltpu.VMEM((CHUNK, 16), jnp.float32),
        pltpu.VMEM_SHARED((N, 16), jnp.float32),
        pltpu.SemaphoreType.DMA,
    ],
)
def _sc_degree(dst_hbm, out_hbm, didx, buf, acc, sem):
    c = lax.axis_index("c")
    s = lax.axis_index("s")
    pltpu.sync_copy(dst_hbm.at[c, s], didx)

    _zero_fill(buf, CHUNK, 16)
    for r in range(ROWS_PT // CHUNK):
        pltpu.sync_copy(buf, acc.at[pl.ds((s * (ROWS_PT // CHUNK) + r) * CHUNK, CHUNK)])

    one16 = jnp.ones((16,), jnp.float32)

    @pl.loop(0, CHUNK)
    def _(i):
        buf[i, pl.ds(0, 16)] = one16

    plsc.subcore_barrier()

    @pl.loop(0, DCHUNK, step=8)
    def _(j0):
        descs = [pltpu.async_copy(buf, acc.at[didx.at[j0 + b]], sem, add=True)
                 for b in range(8)]
        for d in descs:
            d.wait()

    plsc.subcore_barrier()
    for r in range(ROWS_PT // CHUNK):
        pltpu.sync_copy(acc.at[pl.ds(s * ROWS_PT + r * CHUNK, CHUNK)], buf)
        pltpu.sync_copy(buf, out_hbm.at[c, s, r])


@functools.partial(
    pl.kernel,
    out_type=jax.ShapeDtypeStruct((NC, NS, ROWS_PT // CHUNK, CHUNK, HW), jnp.float32),
    mesh=_mesh,
    scratch_types=[
        pltpu.VMEM((NCHUNK // 4, CHUNK), jnp.int32),
        pltpu.VMEM((NCHUNK // 4, CHUNK), jnp.int32),
        pltpu.VMEM((NBUF, CHUNK, HW), jnp.float32),
        pltpu.VMEM((CHUNK, HW), jnp.float32),
        pltpu.VMEM_SHARED((N, HW), jnp.float32),
        pltpu.SemaphoreType.DMA,
        pltpu.SemaphoreType.DMA,
    ],
    compiler_params=pltpu.CompilerParams(use_tc_tiling_on_sc=False),
)
def _sc_scatter(y_hbm, src_hbm, dst_hbm, out_hbm,
                sidx, didx, rows, zbuf, acc, gsem, ssem):
    QCH = NCHUNK // 4
    c = lax.axis_index("c")
    s = lax.axis_index("s")

    _zero_fill(zbuf, CHUNK, HW)
    for r in range(ROWS_PT // CHUNK):
        pltpu.sync_copy(zbuf, acc.at[pl.ds((s * (ROWS_PT // CHUNK) + r) * CHUNK, CHUNK)])
    plsc.subcore_barrier()

    for q in range(4):
        pltpu.sync_copy(src_hbm.at[s, pl.ds(q * QCH, QCH)], sidx)
        pltpu.sync_copy(dst_hbm.at[s, pl.ds(q * QCH, QCH)], didx)

        @pl.loop(0, QCH, step=NBUF)
        def _(j0):
            gds = [pltpu.async_copy(y_hbm.at[c].at[sidx.at[j0 + b]], rows.at[b], gsem)
                   for b in range(NBUF)]
            sds = []
            for b in range(NBUF):
                gds[b].wait()
                sds.append(pltpu.async_copy(rows.at[b], acc.at[didx.at[j0 + b]],
                                            ssem, add=True))
            for d in sds:
                d.wait()

    plsc.subcore_barrier()
    for r in range(ROWS_PT // CHUNK):
        pltpu.sync_copy(acc.at[pl.ds(s * ROWS_PT + r * CHUNK, CHUNK)], zbuf)
        pltpu.sync_copy(zbuf, out_hbm.at[c, s, r])


def _split(h):
    return jnp.stack([h[:, :HW], h[:, HW:]], axis=0)


def _tc1_body(cnt2, x, w1, h1t, dinv):
    cnt = cnt2[0, :, 0:1] + cnt2[1, :, 0:1] + 1.0
    di = lax.rsqrt(cnt)
    dinv[...] = di
    h = jnp.dot(x[...], w1[...], preferred_element_type=jnp.float32) * di
    h1t[...] = _split(h)


def _tc2_body(acc1, h1t, dinv, b1, z1t):
    di = dinv[...][None]
    agg = di * (acc1[...] + h1t[...]) + b1[...][:, None, :]
    z1t[...] = jnp.maximum(agg, 0.0) * di


def _tc3_body(acc2, z1t, dinv, w2, b2, w3, h3t):
    di = dinv[...]
    agg2h = di[None] * (acc2[...] + z1t[...])
    agg2 = jnp.concatenate([agg2h[0], agg2h[1]], axis=1)
    z2 = jnp.maximum(jnp.dot(agg2, w2[...], preferred_element_type=jnp.float32)
                     + b2[...][None, :], 0.0)
    h3 = jnp.dot(z2, w3[...], preferred_element_type=jnp.float32) * di
    h3t[...] = _split(h3)


def _tc4_body(acc3, h3t, dinv, b3, batch, wl1, bl1, wl2, bl2, out):
    di = dinv[...][None]
    z3h = jnp.maximum(di * (acc3[...] + h3t[...]) + b3[...][:, None, :], 0.0)
    z3 = jnp.concatenate([z3h[0], z3h[1]], axis=1)
    gids = lax.broadcasted_iota(jnp.int32, (G, N), 0)
    onehot = (batch[...][None, :] == gids).astype(jnp.float32)
    sums = jnp.dot(onehot, z3, preferred_element_type=jnp.float32)
    cnts = jnp.sum(onehot, axis=1, keepdims=True)
    pooled = sums / jnp.maximum(cnts, 1.0)
    z = jnp.maximum(jnp.dot(pooled, wl1[...], preferred_element_type=jnp.float32)
                    + bl1[...][None, :], 0.0)
    out[...] = jnp.dot(z, wl2[...], preferred_element_type=jnp.float32) + bl2[...][None, :]


def _tc_call(body, out_shapes, *args):
    return pl.pallas_call(body, out_shape=out_shapes)(*args)


def kernel(x, edge_index, batch, W1, b1, W2, b2, W3, b3, Wl1, bl1, Wl2, bl2):
    src = edge_index[0].reshape(NS, NCHUNK, CHUNK)
    dst = edge_index[1].reshape(NS, NCHUNK, CHUNK)
    dst_deg = edge_index[1].reshape(NC, NS, DCHUNK, CHUNK)
    b1s = b1.reshape(NC, HW)
    b3s = b3.reshape(NC, HW)

    cnt2 = _sc_degree(dst_deg).reshape(NC, N, 16)
    h1t, dinv = _tc_call(
        _tc1_body,
        (jax.ShapeDtypeStruct((NC, N, HW), jnp.float32),
         jax.ShapeDtypeStruct((N, 1), jnp.float32)),
        cnt2, x, W1)
    acc1 = _sc_scatter(h1t, src, dst).reshape(NC, N, HW)
    z1t = _tc_call(_tc2_body, jax.ShapeDtypeStruct((NC, N, HW), jnp.float32),
                   acc1, h1t, dinv, b1s)
    acc2 = _sc_scatter(z1t, src, dst).reshape(NC, N, HW)
    h3t = _tc_call(_tc3_body, jax.ShapeDtypeStruct((NC, N, HW), jnp.float32),
                   acc2, z1t, dinv, W2, b2, W3)
    acc3 = _sc_scatter(h3t, src, dst).reshape(NC, N, HW)
    out = _tc_call(_tc4_body, jax.ShapeDtypeStruct((G, 1), jnp.float32),
                   acc3, h3t, dinv, b3s, batch, Wl1, bl1, Wl2, bl2)
    return jnp.squeeze(out, -1)
